# baseline (device time: 28878 ns/iter reference)
import jax
import jax.numpy as jnp
from jax import lax
from jax.experimental import pallas as pl
from jax.experimental.pallas import tpu as pltpu

N_DEV = 4
B = 512
D = 128
B_PER = B // N_DEV
N_PHASES = 4


def kernel(x, Win0, Wout0, Win1, Wout1, Win2, Wout2):
    def body(x_ref, win0_ref, wout0_ref, win1_ref, wout1_ref,
             win2_ref, wout2_ref, out_ref, xg_ref, part_ref,
             send_sems, recv_sems):
        my = lax.axis_index("i")

        barrier_sem = pltpu.get_barrier_semaphore()
        for o in range(1, N_DEV):
            peer = lax.rem(my + o, N_DEV)
            pl.semaphore_signal(barrier_sem, inc=1, device_id=(peer,),
                                device_id_type=pl.DeviceIdType.MESH)
        pl.semaphore_wait(barrier_sem, N_DEV - 1)

        def broadcast(buf_ref, phase):
            rdmas = []
            for o in range(1, N_DEV):
                peer = lax.rem(my + o, N_DEV)
                rdma = pltpu.make_async_remote_copy(
                    src_ref=buf_ref.at[my],
                    dst_ref=buf_ref.at[my],
                    send_sem=send_sems.at[phase, o - 1],
                    recv_sem=recv_sems.at[phase, o - 1],
                    device_id=(peer,),
                    device_id_type=pl.DeviceIdType.MESH,
                )
                rdma.start()
                rdmas.append(rdma)
            for rdma in rdmas:
                rdma.wait()

        xg_ref[my] = x_ref[...].astype(jnp.bfloat16)
        broadcast(xg_ref, 0)
        act = xg_ref[...].reshape(B, D)

        layers = [(win0_ref, wout0_ref), (win1_ref, wout1_ref),
                  (win2_ref, wout2_ref)]
        for k, (win_ref, wout_ref) in enumerate(layers):
            win = win_ref[...].astype(jnp.bfloat16)
            wout = wout_ref[...].astype(jnp.bfloat16)
            h = jnp.maximum(
                lax.dot(act, win, preferred_element_type=jnp.float32), 0.0)
            p = lax.dot(h.astype(jnp.bfloat16), wout,
                        preferred_element_type=jnp.float32)
            part_ref[my] = p.astype(jnp.bfloat16)
            broadcast(part_ref, k + 1)
            acc = part_ref[...].astype(jnp.float32).sum(axis=0)
            if k == len(layers) - 1:
                out_ref[...] = acc
            else:
                act = acc.astype(jnp.bfloat16)

    return pl.pallas_call(
        body,
        out_shape=jax.ShapeDtypeStruct((B, D), jnp.float32),
        in_specs=[pl.BlockSpec(memory_space=pltpu.VMEM)] * 7,
        out_specs=pl.BlockSpec(memory_space=pltpu.VMEM),
        scratch_shapes=[
            pltpu.VMEM((N_DEV, B_PER, D), jnp.bfloat16),
            pltpu.VMEM((N_DEV, B, D), jnp.bfloat16),
            pltpu.SemaphoreType.DMA((N_PHASES, N_DEV - 1)),
            pltpu.SemaphoreType.DMA((N_PHASES, N_DEV - 1)),
        ],
        compiler_params=pltpu.CompilerParams(collective_id=0),
    )(x, Win0, Wout0, Win1, Wout1, Win2, Wout2)
